# Initial kernel scaffold; baseline (speedup 1.0000x reference)
import jax
import jax.numpy as jnp
from jax.experimental import pallas as pl


def kernel(x, edge_index, edge_weight, W1, b1, W2, b2, Wfc, bfc):
    raise NotImplementedError("write your pallas kernel here")


# speedup vs baseline: 24.3320x; 24.3320x over previous
import jax
import jax.numpy as jnp
from jax import lax
from jax.experimental import pallas as pl
from jax.experimental.pallas import tpu as pltpu
from jax.experimental.pallas import tpu_sc as plsc

NC = 2
NS = 16
NW = NC * NS
L = 16

N = 10000
E = 320000
D = 128
H1 = 16
H2 = 32

CH = 128
ETOT = E + N
SB = -(-ETOT // (NW * CH))
EPAD = NW * SB * CH
NPAD = 10240
PT = NPAD // NS


_MESH = plsc.VectorSubcoreMesh(
    core_axis_name="c", subcore_axis_name="s", num_cores=NC, num_subcores=NS
)

_SC_PARAMS = pltpu.CompilerParams(
    needs_layout_passes=False, use_tc_tiling_on_sc=False
)


def _deg_body(col_hbm, ew_hbm, out_hbm, idx_v, ew_v, zero_v, acc):
    c = lax.axis_index("c")
    s = lax.axis_index("s")
    w = c * NS + s

    pltpu.sync_copy(col_hbm.at[w], idx_v)
    pltpu.sync_copy(ew_hbm.at[w], ew_v)

    def _zero(i, _):
        zero_v[pl.ds(i * L, L)] = jnp.zeros((L,), jnp.float32)
        return 0

    lax.fori_loop(0, PT // L, _zero, 0)
    pltpu.sync_copy(zero_v, acc.at[pl.ds(s * PT, PT)])
    plsc.subcore_barrier()

    def _step(j, _):
        pltpu.sync_copy(ew_v.at[j], acc.at[idx_v.at[j]], add=True)
        return 0

    lax.fori_loop(0, SB, _step, 0)
    plsc.subcore_barrier()
    pltpu.sync_copy(acc.at[pl.ds(s * PT, PT)], out_hbm.at[c, pl.ds(s * PT, PT)])


_deg_call = pl.kernel(
    _deg_body,
    out_type=jax.ShapeDtypeStruct((NC, NPAD), jnp.float32),
    mesh=_MESH,
    compiler_params=_SC_PARAMS,
    scratch_types=[
        pltpu.VMEM((SB, CH), jnp.int32),
        pltpu.VMEM((SB, CH), jnp.float32),
        pltpu.VMEM((PT,), jnp.float32),
        pltpu.VMEM_SHARED((NPAD,), jnp.float32),
    ],
)


def _make_agg_call(H):
    def _agg_body(tab_hbm, dinv_hbm, row_hbm, col_hbm, ew_hbm, out_hbm,
                  dinv_v, idxr_v, idxc_v, ew_v, rows_v, norm_v, zero_v, acc,
                  sem):
        c = lax.axis_index("c")
        s = lax.axis_index("s")
        w = c * NS + s

        pltpu.sync_copy(dinv_hbm, dinv_v)
        pltpu.sync_copy(row_hbm.at[w], idxr_v)
        pltpu.sync_copy(col_hbm.at[w], idxc_v)
        pltpu.sync_copy(ew_hbm.at[w], ew_v)

        def _zero(i, _):
            for q in range(H // L):
                zero_v[i, pl.ds(q * L, L)] = jnp.zeros((L,), jnp.float32)
            return 0

        lax.fori_loop(0, PT, _zero, 0)
        pltpu.sync_copy(zero_v, acc.at[pl.ds(s * PT, PT)])
        plsc.subcore_barrier()

        def _chunk(j, _):
            pltpu.async_copy(tab_hbm.at[idxr_v.at[j]], rows_v, sem).wait()
            for g in range(CH // L):
                r16 = idxr_v[j, pl.ds(g * L, L)]
                c16 = idxc_v[j, pl.ds(g * L, L)]
                e16 = ew_v[j, pl.ds(g * L, L)]
                dr = plsc.load_gather(dinv_v, [r16])
                dc = plsc.load_gather(dinv_v, [c16])
                norm_v[pl.ds(g * L, L)] = dr * e16 * dc

            def _scale(r8, _):
                for u in range(8):
                    r = r8 * 8 + u
                    nb = plsc.load_gather(
                        norm_v, [jnp.full((L,), r, jnp.int32)]
                    )
                    for q in range(H // L):
                        rows_v[r, pl.ds(q * L, L)] = (
                            rows_v[r, pl.ds(q * L, L)] * nb
                        )
                return 0

            lax.fori_loop(0, CH // 8, _scale, 0)
            pltpu.sync_copy(rows_v, acc.at[idxc_v.at[j]], add=True)
            return 0

        lax.fori_loop(0, SB, _chunk, 0)
        plsc.subcore_barrier()
        pltpu.sync_copy(acc.at[pl.ds(s * PT, PT)],
                        out_hbm.at[c, pl.ds(s * PT, PT)])

    return pl.kernel(
        _agg_body,
        out_type=jax.ShapeDtypeStruct((NC, NPAD, H), jnp.float32),
        mesh=_MESH,
        compiler_params=_SC_PARAMS,
        scratch_types=[
            pltpu.VMEM((NPAD,), jnp.float32),
            pltpu.VMEM((SB, CH), jnp.int32),
            pltpu.VMEM((SB, CH), jnp.int32),
            pltpu.VMEM((SB, CH), jnp.float32),
            pltpu.VMEM((CH, H), jnp.float32),
            pltpu.VMEM((CH,), jnp.float32),
            pltpu.VMEM((PT, H), jnp.float32),
            pltpu.VMEM_SHARED((NPAD, H), jnp.float32),
            pltpu.SemaphoreType.DMA,
        ],
    )


_agg_call_h1 = _make_agg_call(H1)
_agg_call_h2 = _make_agg_call(H2)


def _lin1_body(x_ref, w_ref, o_ref):
    o_ref[...] = jnp.dot(x_ref[...], w_ref[...],
                         preferred_element_type=jnp.float32)


def _dinv_body(dp_ref, o_ref):
    deg = dp_ref[0] + dp_ref[1]
    o_ref[...] = jnp.where(deg > 0, lax.rsqrt(deg), 0.0)


def _relu_lin_body(p_ref, b_ref, w2_ref, o_ref):
    h = jnp.maximum(p_ref[0] + p_ref[1] + b_ref[...], 0.0)
    o_ref[...] = jnp.dot(h, w2_ref[...], preferred_element_type=jnp.float32)


def _head_body(q_ref, b2_ref, wfc_ref, bfc_ref, o_ref):
    m = q_ref[0] + q_ref[1] + b2_ref[...]
    o_ref[...] = jnp.dot(m, wfc_ref[...],
                         preferred_element_type=jnp.float32) + bfc_ref[...]


def kernel(x, edge_index, edge_weight, W1, b1, W2, b2, Wfc, bfc):
    n = x.shape[0]
    loop = jnp.arange(n, dtype=edge_index.dtype)
    row = jnp.concatenate([edge_index[0], loop])
    col = jnp.concatenate([edge_index[1], loop])
    ew = jnp.concatenate([edge_weight, jnp.ones((n,), edge_weight.dtype)])
    pad = EPAD - ETOT
    row3 = jnp.pad(row, (0, pad)).reshape(NW, SB, CH).astype(jnp.int32)
    col3 = jnp.pad(col, (0, pad)).reshape(NW, SB, CH).astype(jnp.int32)
    ew3 = jnp.pad(ew, (0, pad)).reshape(NW, SB, CH)
    xpad = jnp.pad(x, ((0, NPAD - n), (0, 0)))

    xw1 = pl.pallas_call(
        _lin1_body,
        out_shape=jax.ShapeDtypeStruct((NPAD, H1), jnp.float32),
    )(xpad, W1)

    degp = _deg_call(col3, ew3)
    dinv2d = pl.pallas_call(
        _dinv_body,
        out_shape=jax.ShapeDtypeStruct((NPAD // 128, 128), jnp.float32),
    )(degp.reshape(NC, NPAD // 128, 128))
    dinv = dinv2d.reshape(NPAD)

    p1 = _agg_call_h1(xw1, dinv, row3, col3, ew3)
    xw2 = pl.pallas_call(
        _relu_lin_body,
        out_shape=jax.ShapeDtypeStruct((NPAD, H2), jnp.float32),
    )(p1, b1.reshape(1, H1), W2)

    p2 = _agg_call_h2(xw2, dinv, row3, col3, ew3)

    out = pl.pallas_call(
        _head_body,
        out_shape=jax.ShapeDtypeStruct((NPAD, 1), jnp.float32),
    )(p2, b2.reshape(1, H2), Wfc, bfc.reshape(1, 1))
    return out[:n]


# speedup vs baseline: 33.8003x; 1.3891x over previous
import jax
import jax.numpy as jnp
from jax import lax
from jax.experimental import pallas as pl
from jax.experimental.pallas import tpu as pltpu
from jax.experimental.pallas import tpu_sc as plsc

NC = 2
NS = 16
NW = NC * NS
L = 16

N = 10000
E = 320000
D = 128
H1 = 16
H2 = 32

CH = 128
ETOT = E + N
SB = -(-ETOT // (NW * CH))
EPAD = NW * SB * CH
NPAD = 10240
PT = NPAD // NS


_MESH = plsc.VectorSubcoreMesh(
    core_axis_name="c", subcore_axis_name="s", num_cores=NC, num_subcores=NS
)

_SC_PARAMS = pltpu.CompilerParams(
    needs_layout_passes=False, use_tc_tiling_on_sc=False
)


def _deg_body(col_hbm, ew_hbm, out_hbm, idx_v, ew_v, zero_v, acc):
    c = lax.axis_index("c")
    s = lax.axis_index("s")
    w = c * NS + s

    pltpu.sync_copy(col_hbm.at[w], idx_v)
    pltpu.sync_copy(ew_hbm.at[w], ew_v)

    def _zero(i, _):
        zero_v[pl.ds(i * L, L)] = jnp.zeros((L,), jnp.float32)
        return 0

    lax.fori_loop(0, PT // L, _zero, 0)
    pltpu.sync_copy(zero_v, acc.at[pl.ds(s * PT, PT)])
    plsc.subcore_barrier()

    def _step(j, _):
        pltpu.sync_copy(ew_v.at[j], acc.at[idx_v.at[j]], add=True)
        return 0

    lax.fori_loop(0, SB, _step, 0)
    plsc.subcore_barrier()
    pltpu.sync_copy(acc.at[pl.ds(s * PT, PT)], out_hbm.at[c, pl.ds(s * PT, PT)])


_deg_call = pl.kernel(
    _deg_body,
    out_type=jax.ShapeDtypeStruct((NC, NPAD), jnp.float32),
    mesh=_MESH,
    compiler_params=_SC_PARAMS,
    scratch_types=[
        pltpu.VMEM((SB, CH), jnp.int32),
        pltpu.VMEM((SB, CH), jnp.float32),
        pltpu.VMEM((PT,), jnp.float32),
        pltpu.VMEM_SHARED((NPAD,), jnp.float32),
    ],
)


NB = 3


def _make_agg_call(H):
    def _agg_body(tab_hbm, dinv_hbm, row_hbm, col_hbm, ew_hbm, out_hbm,
                  dinv_v, idxr_v, idxc_v, ew_v,
                  g0, g1, g2, m0, m1, m2, norm_v, zero_v, acc,
                  gs0, gs1, gs2, ss0, ss1, ss2):
        grow = (g0, g1, g2)
        msg = (m0, m1, m2)
        gsem = (gs0, gs1, gs2)
        ssem = (ss0, ss1, ss2)
        c = lax.axis_index("c")
        s = lax.axis_index("s")
        w = c * NS + s

        pltpu.sync_copy(dinv_hbm, dinv_v)
        pltpu.sync_copy(row_hbm.at[w], idxr_v)
        pltpu.sync_copy(col_hbm.at[w], idxc_v)
        pltpu.sync_copy(ew_hbm.at[w], ew_v)

        def _zero(i, _):
            for q in range(H // L):
                zero_v[i, pl.ds(q * L, L)] = jnp.zeros((L,), jnp.float32)
            return 0

        lax.fori_loop(0, PT, _zero, 0)
        pltpu.sync_copy(zero_v, acc.at[pl.ds(s * PT, PT)])
        plsc.subcore_barrier()

        for b in range(NB):
            pltpu.async_copy(tab_hbm.at[idxr_v.at[b]], grow[b], gsem[b])

        def _iter(g, _):
            for b in range(NB):
                j = g * NB + b
                for q in range(CH // L):
                    r16 = idxr_v[j, pl.ds(q * L, L)]
                    c16 = idxc_v[j, pl.ds(q * L, L)]
                    e16 = ew_v[j, pl.ds(q * L, L)]
                    dr = plsc.load_gather(dinv_v, [r16])
                    dc = plsc.load_gather(dinv_v, [c16])
                    norm_v[pl.ds(q * L, L)] = dr * e16 * dc

                pltpu.make_async_copy(tab_hbm.at[idxr_v.at[0]],
                                      grow[b], gsem[b]).wait()

                @pl.when(j >= NB)
                def _():
                    pltpu.make_async_copy(msg[b], acc.at[idxc_v.at[0]],
                                          ssem[b]).wait()

                def _scale(r8, _):
                    for u in range(8):
                        r = r8 * 8 + u
                        nb = plsc.load_gather(
                            norm_v, [jnp.full((L,), r, jnp.int32)]
                        )
                        for q in range(H // L):
                            msg[b][r, pl.ds(q * L, L)] = (
                                grow[b][r, pl.ds(q * L, L)] * nb
                            )
                    return 0

                lax.fori_loop(0, CH // 8, _scale, 0)

                @pl.when(j + NB < SB)
                def _():
                    pltpu.async_copy(tab_hbm.at[idxr_v.at[j + NB]],
                                     grow[b], gsem[b])

                pltpu.async_copy(msg[b], acc.at[idxc_v.at[j]], ssem[b],
                                 add=True)
            return 0

        lax.fori_loop(0, SB // NB, _iter, 0)
        for b in range(NB):
            pltpu.make_async_copy(msg[b], acc.at[idxc_v.at[0]],
                                  ssem[b]).wait()
        plsc.subcore_barrier()
        pltpu.sync_copy(acc.at[pl.ds(s * PT, PT)],
                        out_hbm.at[c, pl.ds(s * PT, PT)])

    return pl.kernel(
        _agg_body,
        out_type=jax.ShapeDtypeStruct((NC, NPAD, H), jnp.float32),
        mesh=_MESH,
        compiler_params=_SC_PARAMS,
        scratch_types=[
            pltpu.VMEM((NPAD,), jnp.float32),
            pltpu.VMEM((SB, CH), jnp.int32),
            pltpu.VMEM((SB, CH), jnp.int32),
            pltpu.VMEM((SB, CH), jnp.float32),
            pltpu.VMEM((CH, H), jnp.float32),
            pltpu.VMEM((CH, H), jnp.float32),
            pltpu.VMEM((CH, H), jnp.float32),
            pltpu.VMEM((CH, H), jnp.float32),
            pltpu.VMEM((CH, H), jnp.float32),
            pltpu.VMEM((CH, H), jnp.float32),
            pltpu.VMEM((CH,), jnp.float32),
            pltpu.VMEM((PT, H), jnp.float32),
            pltpu.VMEM_SHARED((NPAD, H), jnp.float32),
            pltpu.SemaphoreType.DMA,
            pltpu.SemaphoreType.DMA,
            pltpu.SemaphoreType.DMA,
            pltpu.SemaphoreType.DMA,
            pltpu.SemaphoreType.DMA,
            pltpu.SemaphoreType.DMA,
        ],
    )


_agg_call_h1 = _make_agg_call(H1)
_agg_call_h2 = _make_agg_call(H2)


def _lin1_body(x_ref, w_ref, o_ref):
    o_ref[...] = jnp.dot(x_ref[...], w_ref[...],
                         preferred_element_type=jnp.float32)


def _dinv_body(dp_ref, o_ref):
    deg = dp_ref[0] + dp_ref[1]
    o_ref[...] = jnp.where(deg > 0, lax.rsqrt(deg), 0.0)


def _relu_lin_body(p_ref, b_ref, w2_ref, o_ref):
    h = jnp.maximum(p_ref[0] + p_ref[1] + b_ref[...], 0.0)
    o_ref[...] = jnp.dot(h, w2_ref[...], preferred_element_type=jnp.float32)


def _head_body(q_ref, b2_ref, wfc_ref, bfc_ref, o_ref):
    m = q_ref[0] + q_ref[1] + b2_ref[...]
    o_ref[...] = jnp.dot(m, wfc_ref[...],
                         preferred_element_type=jnp.float32) + bfc_ref[...]


def kernel(x, edge_index, edge_weight, W1, b1, W2, b2, Wfc, bfc):
    n = x.shape[0]
    loop = jnp.arange(n, dtype=edge_index.dtype)
    row = jnp.concatenate([edge_index[0], loop])
    col = jnp.concatenate([edge_index[1], loop])
    ew = jnp.concatenate([edge_weight, jnp.ones((n,), edge_weight.dtype)])
    pad = EPAD - ETOT
    row3 = jnp.pad(row, (0, pad)).reshape(NW, SB, CH).astype(jnp.int32)
    col3 = jnp.pad(col, (0, pad)).reshape(NW, SB, CH).astype(jnp.int32)
    ew3 = jnp.pad(ew, (0, pad)).reshape(NW, SB, CH)
    xpad = jnp.pad(x, ((0, NPAD - n), (0, 0)))

    xw1 = pl.pallas_call(
        _lin1_body,
        out_shape=jax.ShapeDtypeStruct((NPAD, H1), jnp.float32),
    )(xpad, W1)

    degp = _deg_call(col3, ew3)
    dinv2d = pl.pallas_call(
        _dinv_body,
        out_shape=jax.ShapeDtypeStruct((NPAD // 128, 128), jnp.float32),
    )(degp.reshape(NC, NPAD // 128, 128))
    dinv = dinv2d.reshape(NPAD)

    p1 = _agg_call_h1(xw1, dinv, row3, col3, ew3)
    xw2 = pl.pallas_call(
        _relu_lin_body,
        out_shape=jax.ShapeDtypeStruct((NPAD, H2), jnp.float32),
    )(p1, b1.reshape(1, H1), W2)

    p2 = _agg_call_h2(xw2, dinv, row3, col3, ew3)

    out = pl.pallas_call(
        _head_body,
        out_shape=jax.ShapeDtypeStruct((NPAD, 1), jnp.float32),
    )(p2, b2.reshape(1, H2), Wfc, bfc.reshape(1, 1))
    return out[:n]


# speedup vs baseline: 35.6177x; 1.0538x over previous
import jax
import jax.numpy as jnp
import numpy as _np
from jax import lax
from jax.experimental import pallas as pl
from jax.experimental.pallas import tpu as pltpu
from jax.experimental.pallas import tpu_sc as plsc

NC = 2
NS = 16
NW = NC * NS
L = 16

N = 10000
E = 320000
D = 128
H1 = 16
H2 = 32

CH = 128
ETOT = E + N
SB = -(-ETOT // (NW * CH))
EPAD = NW * SB * CH
NPAD = 10240
PT = NPAD // NS


_MESH = plsc.VectorSubcoreMesh(
    core_axis_name="c", subcore_axis_name="s", num_cores=NC, num_subcores=NS
)

_SC_PARAMS = pltpu.CompilerParams(
    needs_layout_passes=False, use_tc_tiling_on_sc=False
)


def _deg_body(col_hbm, ew_hbm, out_hbm, idx_v, ew_v, zero_v, acc):
    c = lax.axis_index("c")
    s = lax.axis_index("s")
    w = c * NS + s

    pltpu.sync_copy(col_hbm.at[w], idx_v)
    pltpu.sync_copy(ew_hbm.at[w], ew_v)

    def _zero(i, _):
        zero_v[pl.ds(i * L, L)] = jnp.zeros((L,), jnp.float32)
        return 0

    lax.fori_loop(0, PT // L, _zero, 0)
    pltpu.sync_copy(zero_v, acc.at[pl.ds(s * PT, PT)])
    plsc.subcore_barrier()

    def _step(j, _):
        pltpu.sync_copy(ew_v.at[j], acc.at[idx_v.at[j]], add=True)
        return 0

    lax.fori_loop(0, SB, _step, 0)
    plsc.subcore_barrier()
    pltpu.sync_copy(acc.at[pl.ds(s * PT, PT)], out_hbm.at[c, pl.ds(s * PT, PT)])


_deg_call = pl.kernel(
    _deg_body,
    out_type=jax.ShapeDtypeStruct((NC, NPAD), jnp.float32),
    mesh=_MESH,
    compiler_params=_SC_PARAMS,
    scratch_types=[
        pltpu.VMEM((SB, CH), jnp.int32),
        pltpu.VMEM((SB, CH), jnp.float32),
        pltpu.VMEM((PT,), jnp.float32),
        pltpu.VMEM_SHARED((NPAD,), jnp.float32),
    ],
)


NB = 3

_BCAST_DNUMS = lax.GatherDimensionNumbers(
    offset_dims=(), collapsed_slice_dims=(0,), start_index_map=(0,)
)
def _lane_bcast(vec, u):
    idx = jnp.full((L, 1), u, jnp.int32)
    return lax.gather(vec, idx, _BCAST_DNUMS, slice_sizes=(1,),
                      mode=lax.GatherScatterMode.PROMISE_IN_BOUNDS)


def _make_agg_call(H):
    def _agg_body(tab_hbm, dinv_hbm, row_hbm, col_hbm, ew_hbm, out_hbm,
                  dinv_v, idxr_v, idxc_v, ew_v,
                  g0, g1, g2, m0, m1, m2, zero_v, acc,
                  gs0, gs1, gs2, ss0, ss1, ss2):
        grow = (g0, g1, g2)
        msg = (m0, m1, m2)
        gsem = (gs0, gs1, gs2)
        ssem = (ss0, ss1, ss2)
        c = lax.axis_index("c")
        s = lax.axis_index("s")
        w = c * NS + s

        pltpu.sync_copy(dinv_hbm, dinv_v)
        pltpu.sync_copy(row_hbm.at[w], idxr_v)
        pltpu.sync_copy(col_hbm.at[w], idxc_v)
        pltpu.sync_copy(ew_hbm.at[w], ew_v)

        def _zero(i, _):
            for q in range(H // L):
                zero_v[i, pl.ds(q * L, L)] = jnp.zeros((L,), jnp.float32)
            return 0

        lax.fori_loop(0, PT, _zero, 0)
        pltpu.sync_copy(zero_v, acc.at[pl.ds(s * PT, PT)])
        plsc.subcore_barrier()

        for b in range(NB):
            pltpu.async_copy(tab_hbm.at[idxr_v.at[b]], grow[b], gsem[b])

        def _iter(g, _):
            for b in range(NB):
                j = g * NB + b
                pltpu.make_async_copy(tab_hbm.at[idxr_v.at[0]],
                                      grow[b], gsem[b]).wait()

                @pl.when(j >= NB)
                def _():
                    pltpu.make_async_copy(msg[b], acc.at[idxc_v.at[0]],
                                          ssem[b]).wait()

                def _scale(q, _):
                    r16 = idxr_v[j, pl.ds(q * L, L)]
                    c16 = idxc_v[j, pl.ds(q * L, L)]
                    e16 = ew_v[j, pl.ds(q * L, L)]
                    dr = plsc.load_gather(dinv_v, [r16])
                    dc = plsc.load_gather(dinv_v, [c16])
                    norm16 = dr * e16 * dc
                    for u in range(L):
                        nb = _lane_bcast(norm16, u)
                        r = q * L + u
                        for hq in range(H // L):
                            msg[b][r, pl.ds(hq * L, L)] = (
                                grow[b][r, pl.ds(hq * L, L)] * nb
                            )
                    return 0

                lax.fori_loop(0, CH // L, _scale, 0)

                @pl.when(j + NB < SB)
                def _():
                    pltpu.async_copy(tab_hbm.at[idxr_v.at[j + NB]],
                                     grow[b], gsem[b])

                pltpu.async_copy(msg[b], acc.at[idxc_v.at[j]], ssem[b],
                                 add=True)
            return 0

        lax.fori_loop(0, SB // NB, _iter, 0)
        for b in range(NB):
            pltpu.make_async_copy(msg[b], acc.at[idxc_v.at[0]],
                                  ssem[b]).wait()
        plsc.subcore_barrier()
        pltpu.sync_copy(acc.at[pl.ds(s * PT, PT)],
                        out_hbm.at[c, pl.ds(s * PT, PT)])

    return pl.kernel(
        _agg_body,
        out_type=jax.ShapeDtypeStruct((NC, NPAD, H), jnp.float32),
        mesh=_MESH,
        compiler_params=_SC_PARAMS,
        scratch_types=[
            pltpu.VMEM((NPAD,), jnp.float32),
            pltpu.VMEM((SB, CH), jnp.int32),
            pltpu.VMEM((SB, CH), jnp.int32),
            pltpu.VMEM((SB, CH), jnp.float32),
            pltpu.VMEM((CH, H), jnp.float32),
            pltpu.VMEM((CH, H), jnp.float32),
            pltpu.VMEM((CH, H), jnp.float32),
            pltpu.VMEM((CH, H), jnp.float32),
            pltpu.VMEM((CH, H), jnp.float32),
            pltpu.VMEM((CH, H), jnp.float32),
            pltpu.VMEM((PT, H), jnp.float32),
            pltpu.VMEM_SHARED((NPAD, H), jnp.float32),
            pltpu.SemaphoreType.DMA,
            pltpu.SemaphoreType.DMA,
            pltpu.SemaphoreType.DMA,
            pltpu.SemaphoreType.DMA,
            pltpu.SemaphoreType.DMA,
            pltpu.SemaphoreType.DMA,
        ],
    )


_agg_call_h1 = _make_agg_call(H1)
_agg_call_h2 = _make_agg_call(H2)


def _lin1_body(x_ref, w_ref, o_ref):
    o_ref[...] = jnp.dot(x_ref[...], w_ref[...],
                         preferred_element_type=jnp.float32)


def _dinv_body(dp_ref, o_ref):
    deg = dp_ref[0] + dp_ref[1]
    o_ref[...] = jnp.where(deg > 0, lax.rsqrt(deg), 0.0)


def _relu_lin_body(p_ref, b_ref, w2_ref, o_ref):
    h = jnp.maximum(p_ref[0] + p_ref[1] + b_ref[...], 0.0)
    o_ref[...] = jnp.dot(h, w2_ref[...], preferred_element_type=jnp.float32)


def _head_body(q_ref, b2_ref, wfc_ref, bfc_ref, o_ref):
    m = q_ref[0] + q_ref[1] + b2_ref[...]
    o_ref[...] = jnp.dot(m, wfc_ref[...],
                         preferred_element_type=jnp.float32) + bfc_ref[...]


def kernel(x, edge_index, edge_weight, W1, b1, W2, b2, Wfc, bfc):
    n = x.shape[0]
    loop = jnp.arange(n, dtype=edge_index.dtype)
    row = jnp.concatenate([edge_index[0], loop])
    col = jnp.concatenate([edge_index[1], loop])
    ew = jnp.concatenate([edge_weight, jnp.ones((n,), edge_weight.dtype)])
    pad = EPAD - ETOT
    row3 = jnp.pad(row, (0, pad)).reshape(NW, SB, CH).astype(jnp.int32)
    col3 = jnp.pad(col, (0, pad)).reshape(NW, SB, CH).astype(jnp.int32)
    ew3 = jnp.pad(ew, (0, pad)).reshape(NW, SB, CH)
    xpad = jnp.pad(x, ((0, NPAD - n), (0, 0)))

    xw1 = pl.pallas_call(
        _lin1_body,
        out_shape=jax.ShapeDtypeStruct((NPAD, H1), jnp.float32),
    )(xpad, W1)

    degp = _deg_call(col3, ew3)
    dinv2d = pl.pallas_call(
        _dinv_body,
        out_shape=jax.ShapeDtypeStruct((NPAD // 128, 128), jnp.float32),
    )(degp.reshape(NC, NPAD // 128, 128))
    dinv = dinv2d.reshape(NPAD)

    p1 = _agg_call_h1(xw1, dinv, row3, col3, ew3)
    xw2 = pl.pallas_call(
        _relu_lin_body,
        out_shape=jax.ShapeDtypeStruct((NPAD, H2), jnp.float32),
    )(p1, b1.reshape(1, H1), W2)

    p2 = _agg_call_h2(xw2, dinv, row3, col3, ew3)

    out = pl.pallas_call(
        _head_body,
        out_shape=jax.ShapeDtypeStruct((NPAD, 1), jnp.float32),
    )(p2, b2.reshape(1, H2), Wfc, bfc.reshape(1, 1))
    return out[:n]
